# baseline (speedup 1.0000x reference)
import functools

import numpy as np

import jax
import jax.numpy as jnp
from jax import lax
from jax.experimental import pallas as pl
from jax.experimental.pallas import tpu as pltpu
from jax.experimental.pallas import tpu_sc as plsc

N_NODES = 10000
N_EDGES = 320000
D = 128
H = 8
DH = 16
N_CLASSES = 5000
B = 4096
N_LAYERS = 2

HP = 16
NC = 2
NS = 16
NW = NC * NS
N_PAD = 10240
ROWS_PER_S = N_PAD // NS
CHUNK_A = 400
NCHUNK_A = 26
CHUNK_C = 200
NCHUNK_C = 52
EPW = CHUNK_A * NCHUNK_A
E_PAD = EPW * NW
assert EPW == CHUNK_C * NCHUNK_C


def _sc_scores_body(k_hbm, q_hbm, idx_hbm,
                    ex_hbm, ssum_hbm,
                    idxv, kev, qev, exv, zbuf,
                    ssum_sp,
                    semk, semq):
    c = lax.axis_index("c")
    s = lax.axis_index("s")
    wid = s * NC + c

    zero16 = jnp.zeros((16,), jnp.float32)

    def zb(i, _):
        zbuf[i, :] = zero16
        return 0
    lax.fori_loop(0, ROWS_PER_S // 2, zb, 0)
    for half in range(2):
        pltpu.sync_copy(
            zbuf,
            ssum_sp.at[pl.ds(s * ROWS_PER_S + half * (ROWS_PER_S // 2),
                             ROWS_PER_S // 2)])
    plsc.subcore_barrier()

    base = wid * EPW
    lane = lax.iota(jnp.int32, 16)
    swap8 = lane ^ 8

    def chunk_body(ci, _):
        pltpu.sync_copy(idx_hbm.at[wid].at[ci], idxv)
        cpk = pltpu.async_copy(k_hbm.at[idxv.at[0]], kev, semk)
        cpq = pltpu.async_copy(q_hbm.at[idxv.at[1]], qev, semq)
        cpk.wait()
        cpq.wait()

        def edge_body(e4, _):
            for u in range(4):
                e = e4 * 4 + u
                p = [kev[e, pl.ds(j * 16, 16)] * qev[e, pl.ds(j * 16, 16)]
                     for j in range(D // 16)]
                s1 = [p[0] + p[1], p[2] + p[3], p[4] + p[5], p[6] + p[7]]
                s2 = [s1[0] + s1[1], s1[2] + s1[3]]
                acc = s2[0] + s2[1]
                acc = acc + acc[swap8]
                row = jnp.where(lane < H, acc, 0.0)
                exv[e, :] = jnp.exp(row)
            return 0
        lax.fori_loop(0, CHUNK_A // 4, edge_body, 0)

        pltpu.sync_copy(exv, ex_hbm.at[pl.ds(base + ci * CHUNK_A, CHUNK_A)])
        pltpu.sync_copy(exv, ssum_sp.at[idxv.at[1]], add=True)
        return 0

    lax.fori_loop(0, NCHUNK_A, chunk_body, 0)

    plsc.subcore_barrier()
    pltpu.sync_copy(ssum_sp.at[pl.ds(s * ROWS_PER_S, ROWS_PER_S)],
                    ssum_hbm.at[c].at[pl.ds(s * ROWS_PER_S, ROWS_PER_S)])


_sc_scores = functools.partial(
    pl.kernel,
    out_type=[
        jax.ShapeDtypeStruct((E_PAD, HP), jnp.float32),
        jax.ShapeDtypeStruct((NC, N_PAD, HP), jnp.float32),
    ],
    mesh=plsc.VectorSubcoreMesh(core_axis_name="c", subcore_axis_name="s"),
    compiler_params=pltpu.CompilerParams(use_tc_tiling_on_sc=False),
    scratch_types=[
        pltpu.VMEM((2, CHUNK_A), jnp.int32),
        pltpu.VMEM((CHUNK_A, D), jnp.float32),
        pltpu.VMEM((CHUNK_A, D), jnp.float32),
        pltpu.VMEM((CHUNK_A, HP), jnp.float32),
        pltpu.VMEM((ROWS_PER_S // 2, HP), jnp.float32),
        pltpu.VMEM_SHARED((N_PAD, HP), jnp.float32),
        pltpu.SemaphoreType.DMA,
        pltpu.SemaphoreType.DMA,
    ],
)(_sc_scores_body)


def _sc_msgs_body(v_hbm, ex_hbm, ssum_hbm, idx_hbm,
                  msg_hbm,
                  idxv, vev, exv, ssv,
                  msg_sp,
                  semv, semss, semexl):
    c = lax.axis_index("c")
    s = lax.axis_index("s")
    wid = s * NC + c

    zero16 = jnp.zeros((16,), jnp.float32)

    def zb(i, _):
        for j in range(D // 16):
            vev[i, pl.ds(j * 16, 16)] = zero16
        return 0
    lax.fori_loop(0, CHUNK_C, zb, 0)
    nfull = ROWS_PER_S // CHUNK_C
    for j in range(nfull):
        pltpu.sync_copy(vev,
                        msg_sp.at[pl.ds(s * ROWS_PER_S + j * CHUNK_C, CHUNK_C)])
    rem = ROWS_PER_S - nfull * CHUNK_C
    if rem:
        pltpu.sync_copy(vev.at[pl.ds(0, rem)],
                        msg_sp.at[pl.ds(s * ROWS_PER_S + nfull * CHUNK_C, rem)])
    plsc.subcore_barrier()

    base = wid * EPW

    def chunk_body(ci, _):
        pltpu.sync_copy(idx_hbm.at[wid].at[ci], idxv)
        cpv = pltpu.async_copy(v_hbm.at[idxv.at[0]], vev, semv)
        cps = pltpu.async_copy(ssum_hbm.at[idxv.at[1]], ssv, semss)
        cpe = pltpu.async_copy(
            ex_hbm.at[pl.ds(base + ci * CHUNK_C, CHUNK_C)], exv, semexl)
        cpv.wait()
        cps.wait()
        cpe.wait()

        def attn_body(e4, _):
            for u in range(4):
                e = e4 * 4 + u
                exv[e, :] = exv[e, :] / (ssv[e, :] + 1e-9)
            return 0
        lax.fori_loop(0, CHUNK_C // 4, attn_body, 0)

        def w_body(e2, _):
            for u in range(2):
                e = e2 * 2 + u
                ar = exv[e, :]
                for h in range(H):
                    a = ar[h]
                    vev[e, pl.ds(h * DH, DH)] = vev[e, pl.ds(h * DH, DH)] * a
            return 0
        lax.fori_loop(0, CHUNK_C // 2, w_body, 0)

        pltpu.sync_copy(vev, msg_sp.at[idxv.at[1]], add=True)
        return 0

    lax.fori_loop(0, NCHUNK_C, chunk_body, 0)

    plsc.subcore_barrier()
    pltpu.sync_copy(msg_sp.at[pl.ds(s * ROWS_PER_S, ROWS_PER_S)],
                    msg_hbm.at[c].at[pl.ds(s * ROWS_PER_S, ROWS_PER_S)])


_sc_msgs = functools.partial(
    pl.kernel,
    out_type=jax.ShapeDtypeStruct((NC, N_PAD, D), jnp.float32),
    mesh=plsc.VectorSubcoreMesh(core_axis_name="c", subcore_axis_name="s"),
    compiler_params=pltpu.CompilerParams(use_tc_tiling_on_sc=False),
    scratch_types=[
        pltpu.VMEM((2, CHUNK_C), jnp.int32),
        pltpu.VMEM((CHUNK_C, D), jnp.float32),
        pltpu.VMEM((CHUNK_C, HP), jnp.float32),
        pltpu.VMEM((CHUNK_C, HP), jnp.float32),
        pltpu.VMEM_SHARED((N_PAD, D), jnp.float32),
        pltpu.SemaphoreType.DMA,
        pltpu.SemaphoreType.DMA,
        pltpu.SemaphoreType.DMA,
    ],
)(_sc_msgs_body)


_RB = 1024


def _proj_body(x_ref, w_ref, k_ref, q_ref, v_ref):
    r = jnp.dot(x_ref[...], w_ref[...], preferred_element_type=jnp.float32)
    k_ref[...] = r[:, :D]
    q_ref[...] = r[:, D:2 * D]
    v_ref[...] = r[:, 2 * D:]


def _tc_proj(x, wcat):
    return pl.pallas_call(
        _proj_body,
        grid=(N_PAD // _RB,),
        in_specs=[
            pl.BlockSpec((_RB, D), lambda i: (i, 0)),
            pl.BlockSpec((D, 3 * D), lambda i: (0, 0)),
        ],
        out_specs=[
            pl.BlockSpec((_RB, D), lambda i: (i, 0)),
            pl.BlockSpec((_RB, D), lambda i: (i, 0)),
            pl.BlockSpec((_RB, D), lambda i: (i, 0)),
        ],
        out_shape=[jax.ShapeDtypeStruct((N_PAD, D), jnp.float32)] * 3,
    )(x, wcat)


def _epilogue(m0, m1, x, wa, alpha, ls, lb):
    msg = m0 + m1
    t = jnp.dot(jax.nn.gelu(msg), wa, preferred_element_type=jnp.float32)
    a = alpha[0, 0]
    out = t * a + x * (1.0 - a)
    mu = jnp.mean(out, axis=-1, keepdims=True)
    var = jnp.mean((out - mu) * (out - mu), axis=-1, keepdims=True)
    return (out - mu) / jnp.sqrt(var + 1e-5) * ls + lb


def _epi_proj_body(m0_ref, m1_ref, x_ref, wa_ref, alpha_ref, ls_ref, lb_ref,
                   wcat_ref, x1_ref, k_ref, q_ref, v_ref):
    xln = _epilogue(m0_ref[0], m1_ref[0], x_ref[...], wa_ref[...],
                    alpha_ref, ls_ref[...], lb_ref[...])
    x1_ref[...] = xln
    r = jnp.dot(xln, wcat_ref[...], preferred_element_type=jnp.float32)
    k_ref[...] = r[:, :D]
    q_ref[...] = r[:, D:2 * D]
    v_ref[...] = r[:, 2 * D:]


def _tc_epi_proj(msgp, x, wa, alpha, ls, lb, wcat):
    return pl.pallas_call(
        _epi_proj_body,
        grid=(N_PAD // _RB,),
        in_specs=[
            pl.BlockSpec((1, _RB, D), lambda i: (0, i, 0)),
            pl.BlockSpec((1, _RB, D), lambda i: (1, i, 0)),
            pl.BlockSpec((_RB, D), lambda i: (i, 0)),
            pl.BlockSpec((D, D), lambda i: (0, 0)),
            pl.BlockSpec(memory_space=pltpu.SMEM),
            pl.BlockSpec((1, D), lambda i: (0, 0)),
            pl.BlockSpec((1, D), lambda i: (0, 0)),
            pl.BlockSpec((D, 3 * D), lambda i: (0, 0)),
        ],
        out_specs=[
            pl.BlockSpec((_RB, D), lambda i: (i, 0)),
            pl.BlockSpec((_RB, D), lambda i: (i, 0)),
            pl.BlockSpec((_RB, D), lambda i: (i, 0)),
            pl.BlockSpec((_RB, D), lambda i: (i, 0)),
        ],
        out_shape=[jax.ShapeDtypeStruct((N_PAD, D), jnp.float32)] * 4,
    )(msgp, msgp, x, wa, alpha, ls, lb, wcat)


_CLS_PAD = 5120
_CB = 640


def _epi_body(m0_ref, m1_ref, x_ref, wa_ref, alpha_ref, ls_ref, lb_ref,
              out_ref):
    out_ref[...] = _epilogue(m0_ref[0], m1_ref[0], x_ref[...], wa_ref[...],
                             alpha_ref, ls_ref[...], lb_ref[...])


def _tc_epi_cls(msgp, x, wa, alpha, ls, lb):
    return pl.pallas_call(
        _epi_body,
        grid=(_CLS_PAD // _CB,),
        in_specs=[
            pl.BlockSpec((1, _CB, D), lambda i: (0, i, 0)),
            pl.BlockSpec((1, _CB, D), lambda i: (1, i, 0)),
            pl.BlockSpec((_CB, D), lambda i: (i, 0)),
            pl.BlockSpec((D, D), lambda i: (0, 0)),
            pl.BlockSpec(memory_space=pltpu.SMEM),
            pl.BlockSpec((1, D), lambda i: (0, 0)),
            pl.BlockSpec((1, D), lambda i: (0, 0)),
        ],
        out_specs=pl.BlockSpec((_CB, D), lambda i: (i, 0)),
        out_shape=jax.ShapeDtypeStruct((_CLS_PAD, D), jnp.float32),
    )(msgp, msgp, x, wa, alpha, ls, lb)


def _addsum_body(p_ref, o_ref):
    o_ref[...] = p_ref[0] + p_ref[1]


def _tc_addsum(ssump):
    return pl.pallas_call(
        _addsum_body,
        grid=(4,),
        in_specs=[pl.BlockSpec((2, N_PAD // 4, HP), lambda i: (0, i, 0))],
        out_specs=pl.BlockSpec((N_PAD // 4, HP), lambda i: (i, 0)),
        out_shape=jax.ShapeDtypeStruct((N_PAD, HP), jnp.float32),
    )(ssump)


_LB = 512
_LC = 1024


def _logits_body(e_ref, ak_ref, cls_ref, out_ref):
    a = e_ref[...] * ak_ref[...]
    out_ref[...] = lax.dot_general(
        a, cls_ref[...], (((1,), (1,)), ((), ())),
        preferred_element_type=jnp.float32)


def _tc_logits(emb, ak, cls_emb):
    return pl.pallas_call(
        _logits_body,
        grid=(B // _LB, _CLS_PAD // _LC),
        in_specs=[
            pl.BlockSpec((_LB, D), lambda i, j: (i, 0)),
            pl.BlockSpec((1, D), lambda i, j: (0, 0)),
            pl.BlockSpec((_LC, D), lambda i, j: (j, 0)),
        ],
        out_specs=pl.BlockSpec((_LB, _LC), lambda i, j: (i, j)),
        out_shape=jax.ShapeDtypeStruct((B, _CLS_PAD), jnp.float32),
    )(emb, ak, cls_emb)


def kernel(embeddings, node_emb, attn_kernels, Wk, Wq, Wv, Wa, rel_att,
           rel_msg, rel_pri, skip, ln_scale, ln_bias, edge_index):
    npad = E_PAD - N_EDGES
    src_flat = jnp.concatenate([edge_index[0], jnp.zeros((npad,), jnp.int32)])
    dst_flat = jnp.concatenate(
        [edge_index[1], jnp.full((npad,), N_PAD - 1, jnp.int32)])
    idx_a = jnp.stack([src_flat.reshape(NW, NCHUNK_A, CHUNK_A),
                       dst_flat.reshape(NW, NCHUNK_A, CHUNK_A)], axis=2)
    idx_c = jnp.stack([src_flat.reshape(NW, NCHUNK_C, CHUNK_C),
                       dst_flat.reshape(NW, NCHUNK_C, CHUNK_C)], axis=2)

    def fold(w, r):
        return jnp.einsum('dhk,hke->dhe', w.reshape(D, H, DH), r).reshape(D, D)

    perm = np.empty((D,), dtype=np.int32)
    for d in range(DH):
        for h in range(H):
            perm[d * H + h] = h * DH + d
    perm = jnp.asarray(perm)

    wcats = []
    for l in range(N_LAYERS):
        wk = fold(Wk[l], rel_att[l])
        qscale = (rel_pri[l] / jnp.sqrt(jnp.float32(DH)))
        wq = (Wq[l].reshape(D, H, DH) * qscale[None, :, None]).reshape(D, D)
        wv = fold(Wv[l], rel_msg[l])
        wcats.append(jnp.concatenate([wk[:, perm], wq[:, perm], wv], axis=1))

    alphas = jax.nn.sigmoid(skip).reshape(N_LAYERS, 1, 1)

    x = jnp.concatenate(
        [node_emb, jnp.zeros((N_PAD - N_NODES, D), jnp.float32)])
    k, q, v = _tc_proj(x, wcats[0])
    for l in range(N_LAYERS):
        ex, ssump = _sc_scores(k, q, idx_a)
        ssum = _tc_addsum(ssump)
        msgp = _sc_msgs(v, ex, ssum, idx_c)
        if l + 1 < N_LAYERS:
            x, k, q, v = _tc_epi_proj(msgp, x, Wa[l], alphas[l],
                                      ln_scale[l:l + 1], ln_bias[l:l + 1],
                                      wcats[l + 1])
        else:
            cls_emb = _tc_epi_cls(msgp, x, Wa[l], alphas[l],
                                  ln_scale[l:l + 1], ln_bias[l:l + 1])

    logits = _tc_logits(embeddings, attn_kernels.reshape(1, D), cls_emb)
    return logits[:, :N_CLASSES]


# speedup vs baseline: 2.0938x; 2.0938x over previous
import functools

import numpy as np

import jax
import jax.numpy as jnp
from jax import lax
from jax.experimental import pallas as pl
from jax.experimental.pallas import tpu as pltpu
from jax.experimental.pallas import tpu_sc as plsc

N_NODES = 10000
N_EDGES = 320000
D = 128
H = 8
DH = 16
N_CLASSES = 5000
B = 4096
N_LAYERS = 2

HP = 16
NC = 2
NS = 16
NW = NC * NS
EPW = N_EDGES // NW
CHUNK = 200
NCHUNK = EPW // CHUNK
N_PAD = 10240
ROWS_PER_S = N_PAD // NS


def _sc_scores_body(k_hbm, q_hbm, src_hbm, dst_hbm,
                    ex_hbm, ssum_hbm,
                    srcv, dstv, kev, qev, exv, zbuf,
                    ssum_sp,
                    sem0, sem1):
    c = lax.axis_index("c")
    s = lax.axis_index("s")
    wid = s * NC + c

    zero16 = jnp.zeros((16,), jnp.float32)

    def zb(i, _):
        zbuf[i, :] = zero16
        return 0
    lax.fori_loop(0, ROWS_PER_S, zb, 0)
    pltpu.sync_copy(zbuf, ssum_sp.at[pl.ds(s * ROWS_PER_S, ROWS_PER_S)])
    plsc.subcore_barrier()

    base = wid * EPW

    def chunk_body(ci, _):
        off = base + ci * CHUNK
        pltpu.sync_copy(src_hbm.at[pl.ds(off, CHUNK)], srcv)
        pltpu.sync_copy(dst_hbm.at[pl.ds(off, CHUNK)], dstv)
        cp0 = pltpu.async_copy(k_hbm.at[srcv], kev, sem0)
        cp1 = pltpu.async_copy(q_hbm.at[dstv], qev, sem1)
        cp0.wait()
        cp1.wait()

        lane = lax.iota(jnp.int32, 16)
        swap8 = lane ^ 8

        def edge_body(e4, _):
            for u in range(4):
                e = e4 * 4 + u
                p = [kev[e, pl.ds(j * 16, 16)] * qev[e, pl.ds(j * 16, 16)]
                     for j in range(D // 16)]
                s1 = [p[0] + p[1], p[2] + p[3], p[4] + p[5], p[6] + p[7]]
                s2 = [s1[0] + s1[1], s1[2] + s1[3]]
                acc = s2[0] + s2[1]
                acc = acc + acc[swap8]
                row = jnp.where(lane < H, acc, 0.0)
                exv[e, :] = jnp.exp(row)
            return 0
        lax.fori_loop(0, CHUNK // 4, edge_body, 0)

        pltpu.sync_copy(exv, ex_hbm.at[pl.ds(off, CHUNK)])
        pltpu.sync_copy(exv, ssum_sp.at[dstv], add=True)
        return 0

    lax.fori_loop(0, NCHUNK, chunk_body, 0)

    plsc.subcore_barrier()
    pltpu.sync_copy(ssum_sp.at[pl.ds(s * ROWS_PER_S, ROWS_PER_S)],
                    ssum_hbm.at[c].at[pl.ds(s * ROWS_PER_S, ROWS_PER_S)])


_sc_scores = functools.partial(
    pl.kernel,
    out_type=[
        jax.ShapeDtypeStruct((N_EDGES, HP), jnp.float32),
        jax.ShapeDtypeStruct((NC, N_PAD, HP), jnp.float32),
    ],
    mesh=plsc.VectorSubcoreMesh(core_axis_name="c", subcore_axis_name="s"),
    compiler_params=pltpu.CompilerParams(use_tc_tiling_on_sc=False),
    scratch_types=[
        pltpu.VMEM((CHUNK,), jnp.int32),
        pltpu.VMEM((CHUNK,), jnp.int32),
        pltpu.VMEM((CHUNK, D), jnp.float32),
        pltpu.VMEM((CHUNK, D), jnp.float32),
        pltpu.VMEM((CHUNK, HP), jnp.float32),
        pltpu.VMEM((ROWS_PER_S, HP), jnp.float32),
        pltpu.VMEM_SHARED((N_PAD, HP), jnp.float32),
        pltpu.SemaphoreType.DMA,
        pltpu.SemaphoreType.DMA,
    ],
)(_sc_scores_body)


def _sc_msgs_body(v_hbm, ex_hbm, ssum0_hbm, ssum1_hbm, src_hbm, dst_hbm,
                  msg_hbm,
                  srcv, dstv, vev, exv, p0v, p1v,
                  msg_sp,
                  sem0, sem1, sem2):
    c = lax.axis_index("c")
    s = lax.axis_index("s")
    wid = s * NC + c

    zero16 = jnp.zeros((16,), jnp.float32)

    def zb(i, _):
        for j in range(D // 16):
            vev[i, pl.ds(j * 16, 16)] = zero16
        return 0
    lax.fori_loop(0, CHUNK, zb, 0)
    nfull = ROWS_PER_S // CHUNK
    for j in range(nfull):
        pltpu.sync_copy(vev, msg_sp.at[pl.ds(s * ROWS_PER_S + j * CHUNK, CHUNK)])
    rem = ROWS_PER_S - nfull * CHUNK
    if rem:
        pltpu.sync_copy(vev.at[pl.ds(0, rem)],
                        msg_sp.at[pl.ds(s * ROWS_PER_S + nfull * CHUNK, rem)])
    plsc.subcore_barrier()

    base = wid * EPW

    def chunk_body(ci, _):
        off = base + ci * CHUNK
        pltpu.sync_copy(src_hbm.at[pl.ds(off, CHUNK)], srcv)
        pltpu.sync_copy(dst_hbm.at[pl.ds(off, CHUNK)], dstv)
        pltpu.sync_copy(ex_hbm.at[pl.ds(off, CHUNK)], exv)
        cp0 = pltpu.async_copy(v_hbm.at[srcv], vev, sem0)
        cp1 = pltpu.async_copy(ssum0_hbm.at[dstv], p0v, sem1)
        cp2 = pltpu.async_copy(ssum1_hbm.at[dstv], p1v, sem2)
        cp0.wait()
        cp1.wait()
        cp2.wait()

        def attn_body(e4, _):
            for u in range(4):
                e = e4 * 4 + u
                tot = p0v[e, :] + p1v[e, :]
                exv[e, :] = exv[e, :] / (tot + 1e-9)
            return 0
        lax.fori_loop(0, CHUNK // 4, attn_body, 0)

        def w_body(e2, _):
            for u in range(2):
                e = e2 * 2 + u
                ar = exv[e, :]
                for h in range(H):
                    a = ar[h]
                    vev[e, pl.ds(h * DH, DH)] = vev[e, pl.ds(h * DH, DH)] * a
            return 0
        lax.fori_loop(0, CHUNK // 2, w_body, 0)

        pltpu.sync_copy(vev, msg_sp.at[dstv], add=True)
        return 0

    lax.fori_loop(0, NCHUNK, chunk_body, 0)

    plsc.subcore_barrier()
    pltpu.sync_copy(msg_sp.at[pl.ds(s * ROWS_PER_S, ROWS_PER_S)],
                    msg_hbm.at[c].at[pl.ds(s * ROWS_PER_S, ROWS_PER_S)])


_sc_msgs = functools.partial(
    pl.kernel,
    out_type=jax.ShapeDtypeStruct((NC, N_PAD, D), jnp.float32),
    mesh=plsc.VectorSubcoreMesh(core_axis_name="c", subcore_axis_name="s"),
    compiler_params=pltpu.CompilerParams(use_tc_tiling_on_sc=False),
    scratch_types=[
        pltpu.VMEM((CHUNK,), jnp.int32),
        pltpu.VMEM((CHUNK,), jnp.int32),
        pltpu.VMEM((CHUNK, D), jnp.float32),
        pltpu.VMEM((CHUNK, HP), jnp.float32),
        pltpu.VMEM((CHUNK, HP), jnp.float32),
        pltpu.VMEM((CHUNK, HP), jnp.float32),
        pltpu.VMEM_SHARED((N_PAD, D), jnp.float32),
        pltpu.SemaphoreType.DMA,
        pltpu.SemaphoreType.DMA,
        pltpu.SemaphoreType.DMA,
    ],
)(_sc_msgs_body)


_RB = 1000


def _proj_body(x_ref, w_ref, k_ref, q_ref, v_ref):
    r = jnp.dot(x_ref[...], w_ref[...], preferred_element_type=jnp.float32)
    k_ref[...] = r[:, :D]
    q_ref[...] = r[:, D:2 * D]
    v_ref[...] = r[:, 2 * D:]


def _tc_proj(x, wcat):
    return pl.pallas_call(
        _proj_body,
        grid=(N_NODES // _RB,),
        in_specs=[
            pl.BlockSpec((_RB, D), lambda i: (i, 0)),
            pl.BlockSpec((D, 3 * D), lambda i: (0, 0)),
        ],
        out_specs=[
            pl.BlockSpec((_RB, D), lambda i: (i, 0)),
            pl.BlockSpec((_RB, D), lambda i: (i, 0)),
            pl.BlockSpec((_RB, D), lambda i: (i, 0)),
        ],
        out_shape=[jax.ShapeDtypeStruct((N_NODES, D), jnp.float32)] * 3,
    )(x, wcat)


def _epilogue(m0, m1, x, wa, alpha, ls, lb):
    msg = m0 + m1
    t = jnp.dot(jax.nn.gelu(msg), wa, preferred_element_type=jnp.float32)
    a = alpha[0, 0]
    out = t * a + x * (1.0 - a)
    mu = jnp.mean(out, axis=-1, keepdims=True)
    var = jnp.mean((out - mu) * (out - mu), axis=-1, keepdims=True)
    return (out - mu) / jnp.sqrt(var + 1e-5) * ls + lb


def _epi_proj_body(m0_ref, m1_ref, x_ref, wa_ref, alpha_ref, ls_ref, lb_ref,
                   wcat_ref, x1_ref, k_ref, q_ref, v_ref):
    xln = _epilogue(m0_ref[0], m1_ref[0], x_ref[...], wa_ref[...],
                    alpha_ref, ls_ref[...], lb_ref[...])
    x1_ref[...] = xln
    r = jnp.dot(xln, wcat_ref[...], preferred_element_type=jnp.float32)
    k_ref[...] = r[:, :D]
    q_ref[...] = r[:, D:2 * D]
    v_ref[...] = r[:, 2 * D:]


def _tc_epi_proj(msgp, x, wa, alpha, ls, lb, wcat):
    return pl.pallas_call(
        _epi_proj_body,
        grid=(N_NODES // _RB,),
        in_specs=[
            pl.BlockSpec((1, _RB, D), lambda i: (0, i, 0)),
            pl.BlockSpec((1, _RB, D), lambda i: (1, i, 0)),
            pl.BlockSpec((_RB, D), lambda i: (i, 0)),
            pl.BlockSpec((D, D), lambda i: (0, 0)),
            pl.BlockSpec(memory_space=pltpu.SMEM),
            pl.BlockSpec((1, D), lambda i: (0, 0)),
            pl.BlockSpec((1, D), lambda i: (0, 0)),
            pl.BlockSpec((D, 3 * D), lambda i: (0, 0)),
        ],
        out_specs=[
            pl.BlockSpec((_RB, D), lambda i: (i, 0)),
            pl.BlockSpec((_RB, D), lambda i: (i, 0)),
            pl.BlockSpec((_RB, D), lambda i: (i, 0)),
            pl.BlockSpec((_RB, D), lambda i: (i, 0)),
        ],
        out_shape=[jax.ShapeDtypeStruct((N_NODES, D), jnp.float32)] * 4,
    )(msgp, msgp, x, wa, alpha, ls, lb, wcat)


_CLS_PAD = 5120
_CB = 640


def _epi_body(m0_ref, m1_ref, x_ref, wa_ref, alpha_ref, ls_ref, lb_ref,
              out_ref):
    out_ref[...] = _epilogue(m0_ref[0], m1_ref[0], x_ref[...], wa_ref[...],
                             alpha_ref, ls_ref[...], lb_ref[...])


def _tc_epi_cls(msgp, x, wa, alpha, ls, lb):
    return pl.pallas_call(
        _epi_body,
        grid=(_CLS_PAD // _CB,),
        in_specs=[
            pl.BlockSpec((1, _CB, D), lambda i: (0, i, 0)),
            pl.BlockSpec((1, _CB, D), lambda i: (1, i, 0)),
            pl.BlockSpec((_CB, D), lambda i: (i, 0)),
            pl.BlockSpec((D, D), lambda i: (0, 0)),
            pl.BlockSpec(memory_space=pltpu.SMEM),
            pl.BlockSpec((1, D), lambda i: (0, 0)),
            pl.BlockSpec((1, D), lambda i: (0, 0)),
        ],
        out_specs=pl.BlockSpec((_CB, D), lambda i: (i, 0)),
        out_shape=jax.ShapeDtypeStruct((_CLS_PAD, D), jnp.float32),
    )(msgp, msgp, x, wa, alpha, ls, lb)


_LB = 512
_LC = 1024


def _logits_body(e_ref, ak_ref, cls_ref, out_ref):
    a = e_ref[...] * ak_ref[...]
    out_ref[...] = lax.dot_general(
        a, cls_ref[...], (((1,), (1,)), ((), ())),
        preferred_element_type=jnp.float32)


def _tc_logits(emb, ak, cls_emb):
    return pl.pallas_call(
        _logits_body,
        grid=(B // _LB, _CLS_PAD // _LC),
        in_specs=[
            pl.BlockSpec((_LB, D), lambda i, j: (i, 0)),
            pl.BlockSpec((1, D), lambda i, j: (0, 0)),
            pl.BlockSpec((_LC, D), lambda i, j: (j, 0)),
        ],
        out_specs=pl.BlockSpec((_LB, _LC), lambda i, j: (i, j)),
        out_shape=jax.ShapeDtypeStruct((B, _CLS_PAD), jnp.float32),
    )(emb, ak, cls_emb)


def kernel(embeddings, node_emb, attn_kernels, Wk, Wq, Wv, Wa, rel_att,
           rel_msg, rel_pri, skip, ln_scale, ln_bias, edge_index):
    src = edge_index[0]
    dst = edge_index[1]

    def fold(w, r):
        return jnp.einsum('dhk,hke->dhe', w.reshape(D, H, DH), r).reshape(D, D)

    perm = np.empty((D,), dtype=np.int32)
    for d in range(DH):
        for h in range(H):
            perm[d * H + h] = h * DH + d
    perm = jnp.asarray(perm)

    wcats = []
    for l in range(N_LAYERS):
        wk = fold(Wk[l], rel_att[l])
        qscale = (rel_pri[l] / jnp.sqrt(jnp.float32(DH)))
        wq = (Wq[l].reshape(D, H, DH) * qscale[None, :, None]).reshape(D, D)
        wv = fold(Wv[l], rel_msg[l])
        wcats.append(jnp.concatenate([wk[:, perm], wq[:, perm], wv], axis=1))

    alphas = jax.nn.sigmoid(skip).reshape(N_LAYERS, 1, 1)

    x = node_emb
    k, q, v = _tc_proj(x, wcats[0])
    for l in range(N_LAYERS):
        ex, ssump = _sc_scores(k, q, src, dst)
        msgp = _sc_msgs(v, ex, ssump[0], ssump[1], src, dst)
        if l + 1 < N_LAYERS:
            x, k, q, v = _tc_epi_proj(msgp, x, Wa[l], alphas[l],
                                      ln_scale[l:l + 1], ln_bias[l:l + 1],
                                      wcats[l + 1])
        else:
            cls_emb = _tc_epi_cls(msgp, x, Wa[l], alphas[l],
                                  ln_scale[l:l + 1], ln_bias[l:l + 1])

    logits = _tc_logits(embeddings, attn_kernels.reshape(1, D), cls_emb)
    return logits[:, :N_CLASSES]


# speedup vs baseline: 2.1737x; 1.0382x over previous
import functools

import numpy as np

import jax
import jax.numpy as jnp
from jax import lax
from jax.experimental import pallas as pl
from jax.experimental.pallas import tpu as pltpu
from jax.experimental.pallas import tpu_sc as plsc

N_NODES = 10000
N_EDGES = 320000
D = 128
H = 8
DH = 16
N_CLASSES = 5000
B = 4096
N_LAYERS = 2

HP = 16
NC = 2
NS = 16
NW = NC * NS
EPW = N_EDGES // NW
CHUNK = 200
NCHUNK = EPW // CHUNK
CHUNK_A = 400
NCHUNK_A = EPW // CHUNK_A
N_PAD = 10240
ROWS_PER_S = N_PAD // NS


def _sc_scores_body(k_hbm, q_hbm, src_hbm, dst_hbm,
                    ex_hbm, ssum_hbm,
                    srcv, dstv, kev, qev, exv, zbuf,
                    ssum_sp,
                    sem0, sem1):
    c = lax.axis_index("c")
    s = lax.axis_index("s")
    wid = s * NC + c

    zero16 = jnp.zeros((16,), jnp.float32)

    def zb(i, _):
        zbuf[i, :] = zero16
        return 0
    lax.fori_loop(0, ROWS_PER_S, zb, 0)
    pltpu.sync_copy(zbuf, ssum_sp.at[pl.ds(s * ROWS_PER_S, ROWS_PER_S)])
    plsc.subcore_barrier()

    base = wid * EPW

    def chunk_body(ci, _):
        off = base + ci * CHUNK_A
        pltpu.sync_copy(src_hbm.at[pl.ds(off, CHUNK_A)], srcv)
        pltpu.sync_copy(dst_hbm.at[pl.ds(off, CHUNK_A)], dstv)
        cp0 = pltpu.async_copy(k_hbm.at[srcv], kev, sem0)
        cp1 = pltpu.async_copy(q_hbm.at[dstv], qev, sem1)
        cp0.wait()
        cp1.wait()

        lane = lax.iota(jnp.int32, 16)
        swap8 = lane ^ 8

        def edge_body(e4, _):
            for u in range(4):
                e = e4 * 4 + u
                p = [kev[e, pl.ds(j * 16, 16)] * qev[e, pl.ds(j * 16, 16)]
                     for j in range(D // 16)]
                s1 = [p[0] + p[1], p[2] + p[3], p[4] + p[5], p[6] + p[7]]
                s2 = [s1[0] + s1[1], s1[2] + s1[3]]
                acc = s2[0] + s2[1]
                acc = acc + acc[swap8]
                row = jnp.where(lane < H, acc, 0.0)
                exv[e, :] = jnp.exp(row)
            return 0
        lax.fori_loop(0, CHUNK_A // 4, edge_body, 0)

        pltpu.sync_copy(exv, ex_hbm.at[pl.ds(off, CHUNK_A)])
        pltpu.sync_copy(exv, ssum_sp.at[dstv], add=True)
        return 0

    lax.fori_loop(0, NCHUNK_A, chunk_body, 0)

    plsc.subcore_barrier()
    pltpu.sync_copy(ssum_sp.at[pl.ds(s * ROWS_PER_S, ROWS_PER_S)],
                    ssum_hbm.at[c].at[pl.ds(s * ROWS_PER_S, ROWS_PER_S)])


_sc_scores = functools.partial(
    pl.kernel,
    out_type=[
        jax.ShapeDtypeStruct((N_EDGES, HP), jnp.float32),
        jax.ShapeDtypeStruct((NC, N_PAD, HP), jnp.float32),
    ],
    mesh=plsc.VectorSubcoreMesh(core_axis_name="c", subcore_axis_name="s"),
    compiler_params=pltpu.CompilerParams(use_tc_tiling_on_sc=False),
    scratch_types=[
        pltpu.VMEM((CHUNK_A,), jnp.int32),
        pltpu.VMEM((CHUNK_A,), jnp.int32),
        pltpu.VMEM((CHUNK_A, D), jnp.float32),
        pltpu.VMEM((CHUNK_A, D), jnp.float32),
        pltpu.VMEM((CHUNK_A, HP), jnp.float32),
        pltpu.VMEM((ROWS_PER_S, HP), jnp.float32),
        pltpu.VMEM_SHARED((N_PAD, HP), jnp.float32),
        pltpu.SemaphoreType.DMA,
        pltpu.SemaphoreType.DMA,
    ],
)(_sc_scores_body)


def _sc_msgs_body(v_hbm, ex_hbm, ssum0_hbm, ssum1_hbm, src_hbm, dst_hbm,
                  msg_hbm,
                  srcv, dstv, vev, exv, p0v, p1v,
                  msg_sp,
                  sem0, sem1, sem2):
    c = lax.axis_index("c")
    s = lax.axis_index("s")
    wid = s * NC + c

    zero16 = jnp.zeros((16,), jnp.float32)

    def zb(i, _):
        for j in range(D // 16):
            vev[i, pl.ds(j * 16, 16)] = zero16
        return 0
    lax.fori_loop(0, CHUNK, zb, 0)
    nfull = ROWS_PER_S // CHUNK
    for j in range(nfull):
        pltpu.sync_copy(vev, msg_sp.at[pl.ds(s * ROWS_PER_S + j * CHUNK, CHUNK)])
    rem = ROWS_PER_S - nfull * CHUNK
    if rem:
        pltpu.sync_copy(vev.at[pl.ds(0, rem)],
                        msg_sp.at[pl.ds(s * ROWS_PER_S + nfull * CHUNK, rem)])
    plsc.subcore_barrier()

    base = wid * EPW

    def chunk_body(ci, _):
        off = base + ci * CHUNK
        pltpu.sync_copy(src_hbm.at[pl.ds(off, CHUNK)], srcv)
        pltpu.sync_copy(dst_hbm.at[pl.ds(off, CHUNK)], dstv)
        pltpu.sync_copy(ex_hbm.at[pl.ds(off, CHUNK)], exv)
        cp0 = pltpu.async_copy(v_hbm.at[srcv], vev, sem0)
        cp1 = pltpu.async_copy(ssum0_hbm.at[dstv], p0v, sem1)
        cp2 = pltpu.async_copy(ssum1_hbm.at[dstv], p1v, sem2)
        cp0.wait()
        cp1.wait()
        cp2.wait()

        def attn_body(e4, _):
            for u in range(4):
                e = e4 * 4 + u
                tot = p0v[e, :] + p1v[e, :]
                exv[e, :] = exv[e, :] / (tot + 1e-9)
            return 0
        lax.fori_loop(0, CHUNK // 4, attn_body, 0)

        def w_body(e2, _):
            for u in range(2):
                e = e2 * 2 + u
                ar = exv[e, :]
                for h in range(H):
                    a = ar[h]
                    vev[e, pl.ds(h * DH, DH)] = vev[e, pl.ds(h * DH, DH)] * a
            return 0
        lax.fori_loop(0, CHUNK // 2, w_body, 0)

        pltpu.sync_copy(vev, msg_sp.at[dstv], add=True)
        return 0

    lax.fori_loop(0, NCHUNK, chunk_body, 0)

    plsc.subcore_barrier()
    pltpu.sync_copy(msg_sp.at[pl.ds(s * ROWS_PER_S, ROWS_PER_S)],
                    msg_hbm.at[c].at[pl.ds(s * ROWS_PER_S, ROWS_PER_S)])


_sc_msgs = functools.partial(
    pl.kernel,
    out_type=jax.ShapeDtypeStruct((NC, N_PAD, D), jnp.float32),
    mesh=plsc.VectorSubcoreMesh(core_axis_name="c", subcore_axis_name="s"),
    compiler_params=pltpu.CompilerParams(use_tc_tiling_on_sc=False),
    scratch_types=[
        pltpu.VMEM((CHUNK,), jnp.int32),
        pltpu.VMEM((CHUNK,), jnp.int32),
        pltpu.VMEM((CHUNK, D), jnp.float32),
        pltpu.VMEM((CHUNK, HP), jnp.float32),
        pltpu.VMEM((CHUNK, HP), jnp.float32),
        pltpu.VMEM((CHUNK, HP), jnp.float32),
        pltpu.VMEM_SHARED((N_PAD, D), jnp.float32),
        pltpu.SemaphoreType.DMA,
        pltpu.SemaphoreType.DMA,
        pltpu.SemaphoreType.DMA,
    ],
)(_sc_msgs_body)


_RB = 1000


def _proj_body(x_ref, w_ref, k_ref, q_ref, v_ref):
    r = jnp.dot(x_ref[...], w_ref[...], preferred_element_type=jnp.float32)
    k_ref[...] = r[:, :D]
    q_ref[...] = r[:, D:2 * D]
    v_ref[...] = r[:, 2 * D:]


def _tc_proj(x, wcat):
    return pl.pallas_call(
        _proj_body,
        grid=(N_NODES // _RB,),
        in_specs=[
            pl.BlockSpec((_RB, D), lambda i: (i, 0)),
            pl.BlockSpec((D, 3 * D), lambda i: (0, 0)),
        ],
        out_specs=[
            pl.BlockSpec((_RB, D), lambda i: (i, 0)),
            pl.BlockSpec((_RB, D), lambda i: (i, 0)),
            pl.BlockSpec((_RB, D), lambda i: (i, 0)),
        ],
        out_shape=[jax.ShapeDtypeStruct((N_NODES, D), jnp.float32)] * 3,
    )(x, wcat)


def _epilogue(m0, m1, x, wa, alpha, ls, lb):
    msg = m0 + m1
    t = jnp.dot(jax.nn.gelu(msg), wa, preferred_element_type=jnp.float32)
    a = alpha[0, 0]
    out = t * a + x * (1.0 - a)
    mu = jnp.mean(out, axis=-1, keepdims=True)
    var = jnp.mean((out - mu) * (out - mu), axis=-1, keepdims=True)
    return (out - mu) / jnp.sqrt(var + 1e-5) * ls + lb


def _epi_proj_body(m0_ref, m1_ref, x_ref, wa_ref, alpha_ref, ls_ref, lb_ref,
                   wcat_ref, x1_ref, k_ref, q_ref, v_ref):
    xln = _epilogue(m0_ref[0], m1_ref[0], x_ref[...], wa_ref[...],
                    alpha_ref, ls_ref[...], lb_ref[...])
    x1_ref[...] = xln
    r = jnp.dot(xln, wcat_ref[...], preferred_element_type=jnp.float32)
    k_ref[...] = r[:, :D]
    q_ref[...] = r[:, D:2 * D]
    v_ref[...] = r[:, 2 * D:]


def _tc_epi_proj(msgp, x, wa, alpha, ls, lb, wcat):
    return pl.pallas_call(
        _epi_proj_body,
        grid=(N_NODES // _RB,),
        in_specs=[
            pl.BlockSpec((1, _RB, D), lambda i: (0, i, 0)),
            pl.BlockSpec((1, _RB, D), lambda i: (1, i, 0)),
            pl.BlockSpec((_RB, D), lambda i: (i, 0)),
            pl.BlockSpec((D, D), lambda i: (0, 0)),
            pl.BlockSpec(memory_space=pltpu.SMEM),
            pl.BlockSpec((1, D), lambda i: (0, 0)),
            pl.BlockSpec((1, D), lambda i: (0, 0)),
            pl.BlockSpec((D, 3 * D), lambda i: (0, 0)),
        ],
        out_specs=[
            pl.BlockSpec((_RB, D), lambda i: (i, 0)),
            pl.BlockSpec((_RB, D), lambda i: (i, 0)),
            pl.BlockSpec((_RB, D), lambda i: (i, 0)),
            pl.BlockSpec((_RB, D), lambda i: (i, 0)),
        ],
        out_shape=[jax.ShapeDtypeStruct((N_NODES, D), jnp.float32)] * 4,
    )(msgp, msgp, x, wa, alpha, ls, lb, wcat)


_CLS_PAD = 5120
_CB = 640


def _epi_body(m0_ref, m1_ref, x_ref, wa_ref, alpha_ref, ls_ref, lb_ref,
              out_ref):
    out_ref[...] = _epilogue(m0_ref[0], m1_ref[0], x_ref[...], wa_ref[...],
                             alpha_ref, ls_ref[...], lb_ref[...])


def _tc_epi_cls(msgp, x, wa, alpha, ls, lb):
    return pl.pallas_call(
        _epi_body,
        grid=(_CLS_PAD // _CB,),
        in_specs=[
            pl.BlockSpec((1, _CB, D), lambda i: (0, i, 0)),
            pl.BlockSpec((1, _CB, D), lambda i: (1, i, 0)),
            pl.BlockSpec((_CB, D), lambda i: (i, 0)),
            pl.BlockSpec((D, D), lambda i: (0, 0)),
            pl.BlockSpec(memory_space=pltpu.SMEM),
            pl.BlockSpec((1, D), lambda i: (0, 0)),
            pl.BlockSpec((1, D), lambda i: (0, 0)),
        ],
        out_specs=pl.BlockSpec((_CB, D), lambda i: (i, 0)),
        out_shape=jax.ShapeDtypeStruct((_CLS_PAD, D), jnp.float32),
    )(msgp, msgp, x, wa, alpha, ls, lb)


_LB = 512
_LC = 1024


def _logits_body(e_ref, ak_ref, cls_ref, out_ref):
    a = e_ref[...] * ak_ref[...]
    out_ref[...] = lax.dot_general(
        a, cls_ref[...], (((1,), (1,)), ((), ())),
        preferred_element_type=jnp.float32)


def _tc_logits(emb, ak, cls_emb):
    return pl.pallas_call(
        _logits_body,
        grid=(B // _LB, _CLS_PAD // _LC),
        in_specs=[
            pl.BlockSpec((_LB, D), lambda i, j: (i, 0)),
            pl.BlockSpec((1, D), lambda i, j: (0, 0)),
            pl.BlockSpec((_LC, D), lambda i, j: (j, 0)),
        ],
        out_specs=pl.BlockSpec((_LB, _LC), lambda i, j: (i, j)),
        out_shape=jax.ShapeDtypeStruct((B, _CLS_PAD), jnp.float32),
    )(emb, ak, cls_emb)


def kernel(embeddings, node_emb, attn_kernels, Wk, Wq, Wv, Wa, rel_att,
           rel_msg, rel_pri, skip, ln_scale, ln_bias, edge_index):
    src = edge_index[0]
    dst = edge_index[1]

    def fold(w, r):
        return jnp.einsum('dhk,hke->dhe', w.reshape(D, H, DH), r).reshape(D, D)

    perm = np.empty((D,), dtype=np.int32)
    for d in range(DH):
        for h in range(H):
            perm[d * H + h] = h * DH + d
    perm = jnp.asarray(perm)

    wcats = []
    for l in range(N_LAYERS):
        wk = fold(Wk[l], rel_att[l])
        qscale = (rel_pri[l] / jnp.sqrt(jnp.float32(DH)))
        wq = (Wq[l].reshape(D, H, DH) * qscale[None, :, None]).reshape(D, D)
        wv = fold(Wv[l], rel_msg[l])
        wcats.append(jnp.concatenate([wk[:, perm], wq[:, perm], wv], axis=1))

    alphas = jax.nn.sigmoid(skip).reshape(N_LAYERS, 1, 1)

    x = node_emb
    k, q, v = _tc_proj(x, wcats[0])
    for l in range(N_LAYERS):
        ex, ssump = _sc_scores(k, q, src, dst)
        msgp = _sc_msgs(v, ex, ssump[0], ssump[1], src, dst)
        if l + 1 < N_LAYERS:
            x, k, q, v = _tc_epi_proj(msgp, x, Wa[l], alphas[l],
                                      ln_scale[l:l + 1], ln_bias[l:l + 1],
                                      wcats[l + 1])
        else:
            cls_emb = _tc_epi_cls(msgp, x, Wa[l], alphas[l],
                                  ln_scale[l:l + 1], ln_bias[l:l + 1])

    logits = _tc_logits(embeddings, attn_kernels.reshape(1, D), cls_emb)
    return logits[:, :N_CLASSES]


# speedup vs baseline: 2.2011x; 1.0126x over previous
import functools

import numpy as np

import jax
import jax.numpy as jnp
from jax import lax
from jax.experimental import pallas as pl
from jax.experimental.pallas import tpu as pltpu
from jax.experimental.pallas import tpu_sc as plsc

N_NODES = 10000
N_EDGES = 320000
D = 128
H = 8
DH = 16
N_CLASSES = 5000
B = 4096
N_LAYERS = 2

HP = 16
NC = 2
NS = 16
NW = NC * NS
EPW = N_EDGES // NW
CHUNK = 200
NCHUNK = EPW // CHUNK
CHUNK_A = 400
NCHUNK_A = EPW // CHUNK_A
N_PAD = 10240
ROWS_PER_S = N_PAD // NS


def _sc_scores_body(k_hbm, q_hbm, src_hbm, dst_hbm,
                    ex_hbm, ssum_hbm,
                    srcv, dstv, kev, qev, exv, zbuf,
                    ssum_sp,
                    sem0, sem1):
    c = lax.axis_index("c")
    s = lax.axis_index("s")
    wid = s * NC + c

    zero16 = jnp.zeros((16,), jnp.float32)

    def zb(i, _):
        zbuf[i, :] = zero16
        return 0
    lax.fori_loop(0, ROWS_PER_S, zb, 0)
    pltpu.sync_copy(zbuf, ssum_sp.at[pl.ds(s * ROWS_PER_S, ROWS_PER_S)])
    plsc.subcore_barrier()

    base = wid * EPW

    def chunk_body(ci, _):
        off = base + ci * CHUNK_A
        pltpu.sync_copy(src_hbm.at[pl.ds(off, CHUNK_A)], srcv)
        pltpu.sync_copy(dst_hbm.at[pl.ds(off, CHUNK_A)], dstv)
        cp0 = pltpu.async_copy(k_hbm.at[srcv], kev, sem0)
        cp1 = pltpu.async_copy(q_hbm.at[dstv], qev, sem1)
        cp0.wait()
        cp1.wait()

        lane = lax.iota(jnp.int32, 16)
        swap8 = lane ^ 8

        def edge_body(e4, _):
            for u in range(4):
                e = e4 * 4 + u
                p = [kev[e, pl.ds(j * 16, 16)] * qev[e, pl.ds(j * 16, 16)]
                     for j in range(D // 16)]
                s1 = [p[0] + p[1], p[2] + p[3], p[4] + p[5], p[6] + p[7]]
                s2 = [s1[0] + s1[1], s1[2] + s1[3]]
                acc = s2[0] + s2[1]
                acc = acc + acc[swap8]
                row = jnp.where(lane < H, acc, 0.0)
                exv[e, :] = jnp.exp(row)
            return 0
        lax.fori_loop(0, CHUNK_A // 4, edge_body, 0)

        pltpu.sync_copy(exv, ex_hbm.at[pl.ds(off, CHUNK_A)])
        pltpu.sync_copy(exv, ssum_sp.at[dstv], add=True)
        return 0

    lax.fori_loop(0, NCHUNK_A, chunk_body, 0)

    plsc.subcore_barrier()
    pltpu.sync_copy(ssum_sp.at[pl.ds(s * ROWS_PER_S, ROWS_PER_S)],
                    ssum_hbm.at[c].at[pl.ds(s * ROWS_PER_S, ROWS_PER_S)])


_sc_scores = functools.partial(
    pl.kernel,
    out_type=[
        jax.ShapeDtypeStruct((N_EDGES, HP), jnp.float32),
        jax.ShapeDtypeStruct((NC, N_PAD, HP), jnp.float32),
    ],
    mesh=plsc.VectorSubcoreMesh(core_axis_name="c", subcore_axis_name="s"),
    compiler_params=pltpu.CompilerParams(use_tc_tiling_on_sc=False),
    scratch_types=[
        pltpu.VMEM((CHUNK_A,), jnp.int32),
        pltpu.VMEM((CHUNK_A,), jnp.int32),
        pltpu.VMEM((CHUNK_A, D), jnp.float32),
        pltpu.VMEM((CHUNK_A, D), jnp.float32),
        pltpu.VMEM((CHUNK_A, HP), jnp.float32),
        pltpu.VMEM((ROWS_PER_S, HP), jnp.float32),
        pltpu.VMEM_SHARED((N_PAD, HP), jnp.float32),
        pltpu.SemaphoreType.DMA,
        pltpu.SemaphoreType.DMA,
    ],
)(_sc_scores_body)


def _sc_msgs_body(v_hbm, ex_hbm, ssum_hbm, src_hbm, dst_hbm,
                  msg_hbm,
                  srcv, dstv, vev, exv, ssv,
                  msg_sp,
                  sem0, sem1):
    c = lax.axis_index("c")
    s = lax.axis_index("s")
    wid = s * NC + c

    zero16 = jnp.zeros((16,), jnp.float32)

    def zb(i, _):
        for j in range(D // 16):
            vev[i, pl.ds(j * 16, 16)] = zero16
        return 0
    lax.fori_loop(0, CHUNK, zb, 0)
    nfull = ROWS_PER_S // CHUNK
    for j in range(nfull):
        pltpu.sync_copy(vev, msg_sp.at[pl.ds(s * ROWS_PER_S + j * CHUNK, CHUNK)])
    rem = ROWS_PER_S - nfull * CHUNK
    if rem:
        pltpu.sync_copy(vev.at[pl.ds(0, rem)],
                        msg_sp.at[pl.ds(s * ROWS_PER_S + nfull * CHUNK, rem)])
    plsc.subcore_barrier()

    base = wid * EPW

    def chunk_body(ci, _):
        off = base + ci * CHUNK
        pltpu.sync_copy(src_hbm.at[pl.ds(off, CHUNK)], srcv)
        pltpu.sync_copy(dst_hbm.at[pl.ds(off, CHUNK)], dstv)
        pltpu.sync_copy(ex_hbm.at[pl.ds(off, CHUNK)], exv)
        cp0 = pltpu.async_copy(v_hbm.at[srcv], vev, sem0)
        cp1 = pltpu.async_copy(ssum_hbm.at[dstv], ssv, sem1)
        cp0.wait()
        cp1.wait()

        def attn_body(e4, _):
            for u in range(4):
                e = e4 * 4 + u
                exv[e, :] = exv[e, :] / (ssv[e, :] + 1e-9)
            return 0
        lax.fori_loop(0, CHUNK // 4, attn_body, 0)

        def w_body(e2, _):
            for u in range(2):
                e = e2 * 2 + u
                ar = exv[e, :]
                for h in range(H):
                    a = ar[h]
                    vev[e, pl.ds(h * DH, DH)] = vev[e, pl.ds(h * DH, DH)] * a
            return 0
        lax.fori_loop(0, CHUNK // 2, w_body, 0)

        pltpu.sync_copy(vev, msg_sp.at[dstv], add=True)
        return 0

    lax.fori_loop(0, NCHUNK, chunk_body, 0)

    plsc.subcore_barrier()
    pltpu.sync_copy(msg_sp.at[pl.ds(s * ROWS_PER_S, ROWS_PER_S)],
                    msg_hbm.at[c].at[pl.ds(s * ROWS_PER_S, ROWS_PER_S)])


_sc_msgs = functools.partial(
    pl.kernel,
    out_type=jax.ShapeDtypeStruct((NC, N_PAD, D), jnp.float32),
    mesh=plsc.VectorSubcoreMesh(core_axis_name="c", subcore_axis_name="s"),
    compiler_params=pltpu.CompilerParams(use_tc_tiling_on_sc=False),
    scratch_types=[
        pltpu.VMEM((CHUNK,), jnp.int32),
        pltpu.VMEM((CHUNK,), jnp.int32),
        pltpu.VMEM((CHUNK, D), jnp.float32),
        pltpu.VMEM((CHUNK, HP), jnp.float32),
        pltpu.VMEM((CHUNK, HP), jnp.float32),
        pltpu.VMEM_SHARED((N_PAD, D), jnp.float32),
        pltpu.SemaphoreType.DMA,
        pltpu.SemaphoreType.DMA,
    ],
)(_sc_msgs_body)


_RB = 1000


def _proj_body(x_ref, w_ref, k_ref, q_ref, v_ref):
    r = jnp.dot(x_ref[...], w_ref[...], preferred_element_type=jnp.float32)
    k_ref[...] = r[:, :D]
    q_ref[...] = r[:, D:2 * D]
    v_ref[...] = r[:, 2 * D:]


def _tc_proj(x, wcat):
    return pl.pallas_call(
        _proj_body,
        grid=(N_NODES // _RB,),
        in_specs=[
            pl.BlockSpec((_RB, D), lambda i: (i, 0)),
            pl.BlockSpec((D, 3 * D), lambda i: (0, 0)),
        ],
        out_specs=[
            pl.BlockSpec((_RB, D), lambda i: (i, 0)),
            pl.BlockSpec((_RB, D), lambda i: (i, 0)),
            pl.BlockSpec((_RB, D), lambda i: (i, 0)),
        ],
        out_shape=[jax.ShapeDtypeStruct((N_NODES, D), jnp.float32)] * 3,
    )(x, wcat)


def _epilogue(m0, m1, x, wa, alpha, ls, lb):
    msg = m0 + m1
    t = jnp.dot(jax.nn.gelu(msg), wa, preferred_element_type=jnp.float32)
    a = alpha[0, 0]
    out = t * a + x * (1.0 - a)
    mu = jnp.mean(out, axis=-1, keepdims=True)
    var = jnp.mean((out - mu) * (out - mu), axis=-1, keepdims=True)
    return (out - mu) / jnp.sqrt(var + 1e-5) * ls + lb


def _epi_proj_body(m0_ref, m1_ref, x_ref, wa_ref, alpha_ref, ls_ref, lb_ref,
                   wcat_ref, x1_ref, k_ref, q_ref, v_ref):
    xln = _epilogue(m0_ref[0], m1_ref[0], x_ref[...], wa_ref[...],
                    alpha_ref, ls_ref[...], lb_ref[...])
    x1_ref[...] = xln
    r = jnp.dot(xln, wcat_ref[...], preferred_element_type=jnp.float32)
    k_ref[...] = r[:, :D]
    q_ref[...] = r[:, D:2 * D]
    v_ref[...] = r[:, 2 * D:]


def _tc_epi_proj(msgp, x, wa, alpha, ls, lb, wcat):
    return pl.pallas_call(
        _epi_proj_body,
        grid=(N_NODES // _RB,),
        in_specs=[
            pl.BlockSpec((1, _RB, D), lambda i: (0, i, 0)),
            pl.BlockSpec((1, _RB, D), lambda i: (1, i, 0)),
            pl.BlockSpec((_RB, D), lambda i: (i, 0)),
            pl.BlockSpec((D, D), lambda i: (0, 0)),
            pl.BlockSpec(memory_space=pltpu.SMEM),
            pl.BlockSpec((1, D), lambda i: (0, 0)),
            pl.BlockSpec((1, D), lambda i: (0, 0)),
            pl.BlockSpec((D, 3 * D), lambda i: (0, 0)),
        ],
        out_specs=[
            pl.BlockSpec((_RB, D), lambda i: (i, 0)),
            pl.BlockSpec((_RB, D), lambda i: (i, 0)),
            pl.BlockSpec((_RB, D), lambda i: (i, 0)),
            pl.BlockSpec((_RB, D), lambda i: (i, 0)),
        ],
        out_shape=[jax.ShapeDtypeStruct((N_NODES, D), jnp.float32)] * 4,
    )(msgp, msgp, x, wa, alpha, ls, lb, wcat)


_CLS_PAD = 5120
_CB = 640


def _epi_body(m0_ref, m1_ref, x_ref, wa_ref, alpha_ref, ls_ref, lb_ref,
              out_ref):
    out_ref[...] = _epilogue(m0_ref[0], m1_ref[0], x_ref[...], wa_ref[...],
                             alpha_ref, ls_ref[...], lb_ref[...])


def _tc_epi_cls(msgp, x, wa, alpha, ls, lb):
    return pl.pallas_call(
        _epi_body,
        grid=(_CLS_PAD // _CB,),
        in_specs=[
            pl.BlockSpec((1, _CB, D), lambda i: (0, i, 0)),
            pl.BlockSpec((1, _CB, D), lambda i: (1, i, 0)),
            pl.BlockSpec((_CB, D), lambda i: (i, 0)),
            pl.BlockSpec((D, D), lambda i: (0, 0)),
            pl.BlockSpec(memory_space=pltpu.SMEM),
            pl.BlockSpec((1, D), lambda i: (0, 0)),
            pl.BlockSpec((1, D), lambda i: (0, 0)),
        ],
        out_specs=pl.BlockSpec((_CB, D), lambda i: (i, 0)),
        out_shape=jax.ShapeDtypeStruct((_CLS_PAD, D), jnp.float32),
    )(msgp, msgp, x, wa, alpha, ls, lb)


def _addsum_body(p_ref, o_ref):
    o_ref[...] = p_ref[0] + p_ref[1]


def _tc_addsum(ssump):
    return pl.pallas_call(
        _addsum_body,
        grid=(4,),
        in_specs=[pl.BlockSpec((2, N_PAD // 4, HP), lambda i: (0, i, 0))],
        out_specs=pl.BlockSpec((N_PAD // 4, HP), lambda i: (i, 0)),
        out_shape=jax.ShapeDtypeStruct((N_PAD, HP), jnp.float32),
    )(ssump)


_LB = 512
_LC = 1024


def _logits_body(e_ref, ak_ref, cls_ref, out_ref):
    a = e_ref[...] * ak_ref[...]
    out_ref[...] = lax.dot_general(
        a, cls_ref[...], (((1,), (1,)), ((), ())),
        preferred_element_type=jnp.float32)


def _tc_logits(emb, ak, cls_emb):
    return pl.pallas_call(
        _logits_body,
        grid=(B // _LB, _CLS_PAD // _LC),
        in_specs=[
            pl.BlockSpec((_LB, D), lambda i, j: (i, 0)),
            pl.BlockSpec((1, D), lambda i, j: (0, 0)),
            pl.BlockSpec((_LC, D), lambda i, j: (j, 0)),
        ],
        out_specs=pl.BlockSpec((_LB, _LC), lambda i, j: (i, j)),
        out_shape=jax.ShapeDtypeStruct((B, _CLS_PAD), jnp.float32),
    )(emb, ak, cls_emb)


def kernel(embeddings, node_emb, attn_kernels, Wk, Wq, Wv, Wa, rel_att,
           rel_msg, rel_pri, skip, ln_scale, ln_bias, edge_index):
    src = edge_index[0]
    dst = edge_index[1]

    def fold(w, r):
        return jnp.einsum('dhk,hke->dhe', w.reshape(D, H, DH), r).reshape(D, D)

    perm = np.empty((D,), dtype=np.int32)
    for d in range(DH):
        for h in range(H):
            perm[d * H + h] = h * DH + d
    perm = jnp.asarray(perm)

    wcats = []
    for l in range(N_LAYERS):
        wk = fold(Wk[l], rel_att[l])
        qscale = (rel_pri[l] / jnp.sqrt(jnp.float32(DH)))
        wq = (Wq[l].reshape(D, H, DH) * qscale[None, :, None]).reshape(D, D)
        wv = fold(Wv[l], rel_msg[l])
        wcats.append(jnp.concatenate([wk[:, perm], wq[:, perm], wv], axis=1))

    alphas = jax.nn.sigmoid(skip).reshape(N_LAYERS, 1, 1)

    x = node_emb
    k, q, v = _tc_proj(x, wcats[0])
    for l in range(N_LAYERS):
        ex, ssump = _sc_scores(k, q, src, dst)
        ssum = _tc_addsum(ssump)
        msgp = _sc_msgs(v, ex, ssum, src, dst)
        if l + 1 < N_LAYERS:
            x, k, q, v = _tc_epi_proj(msgp, x, Wa[l], alphas[l],
                                      ln_scale[l:l + 1], ln_bias[l:l + 1],
                                      wcats[l + 1])
        else:
            cls_emb = _tc_epi_cls(msgp, x, Wa[l], alphas[l],
                                  ln_scale[l:l + 1], ln_bias[l:l + 1])

    logits = _tc_logits(embeddings, attn_kernels.reshape(1, D), cls_emb)
    return logits[:, :N_CLASSES]
